# Initial kernel scaffold; baseline (speedup 1.0000x reference)
import functools

import jax
import jax.numpy as jnp
from jax import lax
from jax.experimental import pallas as pl
from jax.experimental.pallas import tpu as pltpu
from jax.experimental.pallas import tpu_sc as plsc

N = 10000
E = 320000
D = 128

NC = 2
NS = 16
NW = NC * NS
K = 128
CH = 80
E_PAD = NW * CH * K
N_PAD = 10240
PAD_ROWS = N_PAD - N
RPT = N_PAD // NS

_mesh = plsc.VectorSubcoreMesh(core_axis_name="c", subcore_axis_name="s")


@functools.partial(
    pl.kernel,
    out_type=jax.ShapeDtypeStruct((NC, N_PAD, 16), jnp.float32),
    mesh=_mesh,
    scratch_types=[
        pltpu.VMEM((CH, K), jnp.int32),
        pltpu.VMEM((K, 16), jnp.float32),
    ],
)
def _sc_degree(didx_hbm, ones_hbm, zeros16_hbm, out_hbm, didx_v, ones_v):
    c = lax.axis_index("c")
    s = lax.axis_index("s")
    w = c * NS + s
    base = s * RPT
    pltpu.sync_copy(didx_hbm.at[w], didx_v)
    pltpu.sync_copy(ones_hbm, ones_v)

    def body(dsh):
        pltpu.sync_copy(zeros16_hbm, dsh.at[pl.ds(base, RPT)])
        plsc.subcore_barrier()

        @pl.loop(0, CH)
        def _(ci):
            pltpu.sync_copy(ones_v, dsh.at[didx_v.at[ci]], add=True)

        plsc.subcore_barrier()
        pltpu.sync_copy(dsh.at[pl.ds(base, RPT)],
                        out_hbm.at[c].at[pl.ds(base, RPT)])

    pl.run_scoped(body, pltpu.VMEM_SHARED((N_PAD, 16), jnp.float32))


@functools.partial(
    pl.kernel,
    out_type=jax.ShapeDtypeStruct((NC, N_PAD, D), jnp.float32),
    mesh=_mesh,
    scratch_types=[
        pltpu.VMEM((CH, K), jnp.int32),
        pltpu.VMEM((CH, K), jnp.int32),
        pltpu.VMEM((K, D), jnp.float32),
        pltpu.VMEM((K, D), jnp.float32),
        pltpu.SemaphoreType.DMA,
        pltpu.SemaphoreType.DMA,
    ],
)
def _sc_aggregate(y_hbm, sidx_hbm, didx_hbm, zeros_hbm, out_hbm,
                  sidx_v, didx_v, g0, g1, s0, s1):
    c = lax.axis_index("c")
    s = lax.axis_index("s")
    w = c * NS + s
    base = s * RPT
    pltpu.sync_copy(sidx_hbm.at[w], sidx_v)
    pltpu.sync_copy(didx_hbm.at[w], didx_v)

    def body(zsh):
        @pl.when(c == 0)
        def _():
            pltpu.sync_copy(y_hbm.at[pl.ds(base, RPT)],
                            zsh.at[pl.ds(base, RPT)])

        @pl.when(c == 1)
        def _():
            pltpu.sync_copy(zeros_hbm, zsh.at[pl.ds(base, RPT)])

        plsc.subcore_barrier()

        pltpu.make_async_copy(y_hbm.at[sidx_v.at[0]], g0, s0).start()
        pltpu.make_async_copy(y_hbm.at[sidx_v.at[1]], g1, s1).start()

        @pl.loop(0, CH, step=2)
        def _(ci):
            pltpu.make_async_copy(y_hbm.at[sidx_v.at[ci]], g0, s0).wait()
            pltpu.sync_copy(g0, zsh.at[didx_v.at[ci]], add=True)

            @pl.when(ci + 2 < CH)
            def _():
                pltpu.make_async_copy(
                    y_hbm.at[sidx_v.at[ci + 2]], g0, s0).start()

            pltpu.make_async_copy(y_hbm.at[sidx_v.at[ci + 1]], g1, s1).wait()
            pltpu.sync_copy(g1, zsh.at[didx_v.at[ci + 1]], add=True)

            @pl.when(ci + 3 < CH)
            def _():
                pltpu.make_async_copy(
                    y_hbm.at[sidx_v.at[ci + 3]], g1, s1).start()

        plsc.subcore_barrier()
        pltpu.sync_copy(zsh.at[pl.ds(base, RPT)],
                        out_hbm.at[c].at[pl.ds(base, RPT)])

    pl.run_scoped(body, pltpu.VMEM_SHARED((N_PAD, D), jnp.float32))


_BLK = 1024
_GRID = N_PAD // _BLK


def _mm_body(x_ref, w_ref, o_ref):
    o_ref[...] = jnp.dot(x_ref[...], w_ref[...],
                         preferred_element_type=jnp.float32)


def _tc_matmul(x, w):
    return pl.pallas_call(
        _mm_body,
        grid=(_GRID,),
        in_specs=[
            pl.BlockSpec((_BLK, D), lambda i: (i, 0)),
            pl.BlockSpec((D, D), lambda i: (0, 0)),
        ],
        out_specs=pl.BlockSpec((_BLK, D), lambda i: (i, 0)),
        out_shape=jax.ShapeDtypeStruct((N_PAD, D), jnp.float32),
    )(x, w)


def _scale_body(xw_ref, deg_ref, y_ref, dinv_ref):
    deg = deg_ref[0, :, 0:1] + deg_ref[1, :, 0:1] + 1.0
    dv = lax.rsqrt(deg)
    dinv_ref[...] = dv
    y_ref[...] = xw_ref[...] * dv


def _tc_scale(xw, degp):
    return pl.pallas_call(
        _scale_body,
        grid=(_GRID,),
        in_specs=[
            pl.BlockSpec((_BLK, D), lambda i: (i, 0)),
            pl.BlockSpec((NC, _BLK, 16), lambda i: (0, i, 0)),
        ],
        out_specs=[
            pl.BlockSpec((_BLK, D), lambda i: (i, 0)),
            pl.BlockSpec((_BLK, 1), lambda i: (i, 0)),
        ],
        out_shape=[
            jax.ShapeDtypeStruct((N_PAD, D), jnp.float32),
            jax.ShapeDtypeStruct((N_PAD, 1), jnp.float32),
        ],
    )(xw, degp)


def _mid_body(z_ref, dinv_ref, b_ref, w_ref, y2_ref):
    dv = dinv_ref[...]
    h = (z_ref[0] + z_ref[1]) * dv + b_ref[...]
    h = jnp.maximum(h, 0.0)
    y2_ref[...] = jnp.dot(h, w_ref[...],
                          preferred_element_type=jnp.float32) * dv


def _tc_mid(z, dinv, b, w):
    return pl.pallas_call(
        _mid_body,
        grid=(_GRID,),
        in_specs=[
            pl.BlockSpec((NC, _BLK, D), lambda i: (0, i, 0)),
            pl.BlockSpec((_BLK, 1), lambda i: (i, 0)),
            pl.BlockSpec((1, D), lambda i: (0, 0)),
            pl.BlockSpec((D, D), lambda i: (0, 0)),
        ],
        out_specs=pl.BlockSpec((_BLK, D), lambda i: (i, 0)),
        out_shape=jax.ShapeDtypeStruct((N_PAD, D), jnp.float32),
    )(z, dinv, b, w)


_FBLK = 2000
_FGRID = N // _FBLK


def _final_body(z_ref, dinv_ref, b_ref, o_ref):
    o_ref[...] = (z_ref[0] + z_ref[1]) * dinv_ref[...] + b_ref[...]


def _tc_final(z, dinv, b):
    return pl.pallas_call(
        _final_body,
        grid=(_FGRID,),
        in_specs=[
            pl.BlockSpec((NC, _FBLK, D), lambda i: (0, i, 0)),
            pl.BlockSpec((_FBLK, 1), lambda i: (i, 0)),
            pl.BlockSpec((1, D), lambda i: (0, 0)),
        ],
        out_specs=pl.BlockSpec((_FBLK, D), lambda i: (i, 0)),
        out_shape=jax.ShapeDtypeStruct((N, D), jnp.float32),
    )(z, dinv, b)


def kernel(x, edge_index, W1, b1, W2, b2):
    src = edge_index[0].astype(jnp.int32)
    dst = edge_index[1].astype(jnp.int32)
    pad_idx = N + (jnp.arange(E_PAD - E, dtype=jnp.int32) % PAD_ROWS)
    srcp = jnp.concatenate([src, pad_idx]).reshape(NW, CH, K)
    dstp = jnp.concatenate([dst, pad_idx]).reshape(NW, CH, K)
    x_pad = jnp.pad(x, ((0, N_PAD - N), (0, 0)))
    ones16 = jnp.ones((K, 16), jnp.float32)
    zeros16 = jnp.zeros((RPT, 16), jnp.float32)
    zrows = jnp.zeros((RPT, D), jnp.float32)
    b1r = b1.reshape(1, D)
    b2r = b2.reshape(1, D)

    degp = _sc_degree(dstp, ones16, zeros16)
    xw1 = _tc_matmul(x_pad, W1)
    y1, dinv = _tc_scale(xw1, degp)
    z1 = _sc_aggregate(y1, srcp, dstp, zrows)
    y2 = _tc_mid(z1, dinv, b1r, W2)
    z2 = _sc_aggregate(y2, srcp, dstp, zrows)
    return _tc_final(z2, dinv, b2r)


# speedup vs baseline: 26.0441x; 26.0441x over previous
import dataclasses
import functools

import jax
import jax.numpy as jnp
from jax import lax
from jax.experimental import pallas as pl
from jax.experimental.pallas import tpu as pltpu
from jax.experimental.pallas import tpu_sc as plsc

N = 10000
E = 320000
D = 128

NC = 2
NS = 16
NW = NC * NS
K = 128
CH = 80
E_PAD = NW * CH * K
N_PAD = 10240
PAD_ROWS = N_PAD - N
RPT = N_PAD // NS

_mesh = plsc.VectorSubcoreMesh(core_axis_name="c", subcore_axis_name="s")

_sc_cp = pltpu.CompilerParams()
if "needs_layout_passes" in pltpu.CompilerParams.__dataclass_fields__:
    _sc_cp = dataclasses.replace(_sc_cp, needs_layout_passes=False)
_sc_linear_cp = pltpu.CompilerParams(use_tc_tiling_on_sc=False)


@functools.partial(
    pl.kernel,
    out_type=jax.ShapeDtypeStruct((NC, N_PAD, 16), jnp.float32),
    mesh=_mesh,
    scratch_types=[
        pltpu.VMEM((CH, K), jnp.int32),
        pltpu.VMEM((K, 16), jnp.float32),
        pltpu.VMEM_SHARED((N_PAD, 16), jnp.float32),
    ],
    compiler_params=_sc_linear_cp,
)
def _sc_degree(didx_hbm, ones_hbm, zeros16_hbm, out_hbm, didx_v, ones_v, dsh):
    c = lax.axis_index("c")
    s = lax.axis_index("s")
    w = c * NS + s
    base = s * RPT
    pltpu.sync_copy(didx_hbm.at[w], didx_v)
    pltpu.sync_copy(ones_hbm, ones_v)

    pltpu.sync_copy(zeros16_hbm, dsh.at[pl.ds(base, RPT)])
    plsc.subcore_barrier()

    @pl.loop(0, CH)
    def _(ci):
        pltpu.sync_copy(ones_v, dsh.at[didx_v.at[ci]], add=True)

    plsc.subcore_barrier()
    pltpu.sync_copy(dsh.at[pl.ds(base, RPT)],
                    out_hbm.at[c].at[pl.ds(base, RPT)])


DH = D // NC
CH2 = CH * 2


@functools.partial(
    pl.kernel,
    out_type=jax.ShapeDtypeStruct((NC, N_PAD, DH), jnp.float32),
    mesh=_mesh,
    scratch_types=[
        pltpu.VMEM((CH2, K), jnp.int32),
        pltpu.VMEM((CH2, K), jnp.int32),
        pltpu.VMEM((K, DH), jnp.float32),
        pltpu.VMEM((K, DH), jnp.float32),
        pltpu.VMEM_SHARED((N_PAD, DH), jnp.float32),
        pltpu.SemaphoreType.DMA,
        pltpu.SemaphoreType.DMA,
    ],
    compiler_params=_sc_linear_cp,
)
def _sc_aggregate(y_hbm, sidx_hbm, didx_hbm, out_hbm,
                  sidx_v, didx_v, g0, g1, zsh, s0, s1):
    c = lax.axis_index("c")
    s = lax.axis_index("s")
    base = s * RPT
    yc = y_hbm.at[c]
    pltpu.sync_copy(sidx_hbm.at[2 * s], sidx_v.at[pl.ds(0, CH)])
    pltpu.sync_copy(sidx_hbm.at[2 * s + 1], sidx_v.at[pl.ds(CH, CH)])
    pltpu.sync_copy(didx_hbm.at[2 * s], didx_v.at[pl.ds(0, CH)])
    pltpu.sync_copy(didx_hbm.at[2 * s + 1], didx_v.at[pl.ds(CH, CH)])

    pltpu.sync_copy(yc.at[pl.ds(base, RPT)], zsh.at[pl.ds(base, RPT)])
    plsc.subcore_barrier()

    pltpu.make_async_copy(yc.at[sidx_v.at[0]], g0, s0).start()
    pltpu.make_async_copy(yc.at[sidx_v.at[1]], g1, s1).start()

    @pl.loop(0, CH2, step=2)
    def _(ci):
        pltpu.make_async_copy(yc.at[sidx_v.at[ci]], g0, s0).wait()
        pltpu.sync_copy(g0, zsh.at[didx_v.at[ci]], add=True)

        @pl.when(ci + 2 < CH2)
        def _():
            pltpu.make_async_copy(yc.at[sidx_v.at[ci + 2]], g0, s0).start()

        pltpu.make_async_copy(yc.at[sidx_v.at[ci + 1]], g1, s1).wait()
        pltpu.sync_copy(g1, zsh.at[didx_v.at[ci + 1]], add=True)

        @pl.when(ci + 3 < CH2)
        def _():
            pltpu.make_async_copy(yc.at[sidx_v.at[ci + 3]], g1, s1).start()

    plsc.subcore_barrier()
    pltpu.sync_copy(zsh.at[pl.ds(base, RPT)],
                    out_hbm.at[c].at[pl.ds(base, RPT)])


_BLK = 1024
_GRID = N_PAD // _BLK


def _mm_body(x_ref, w_ref, o_ref):
    o_ref[...] = jnp.dot(x_ref[...], w_ref[...],
                         preferred_element_type=jnp.float32)


def _tc_matmul(x, w):
    return pl.pallas_call(
        _mm_body,
        grid=(_GRID,),
        in_specs=[
            pl.BlockSpec((_BLK, D), lambda i: (i, 0)),
            pl.BlockSpec((D, D), lambda i: (0, 0)),
        ],
        out_specs=pl.BlockSpec((_BLK, D), lambda i: (i, 0)),
        out_shape=jax.ShapeDtypeStruct((N_PAD, D), jnp.float32),
    )(x, w)


def _split(t):
    return jnp.stack([t[:, :DH], t[:, DH:]], axis=0)


def _scale_body(xw_ref, deg_ref, y_ref, dinv_ref):
    deg = deg_ref[0, :, 0:1] + deg_ref[1, :, 0:1] + 1.0
    dv = lax.rsqrt(deg)
    dinv_ref[...] = dv
    y_ref[...] = _split(xw_ref[...] * dv)


def _tc_scale(xw, degp):
    return pl.pallas_call(
        _scale_body,
        grid=(_GRID,),
        in_specs=[
            pl.BlockSpec((_BLK, D), lambda i: (i, 0)),
            pl.BlockSpec((NC, _BLK, 16), lambda i: (0, i, 0)),
        ],
        out_specs=[
            pl.BlockSpec((NC, _BLK, DH), lambda i: (0, i, 0)),
            pl.BlockSpec((_BLK, 1), lambda i: (i, 0)),
        ],
        out_shape=[
            jax.ShapeDtypeStruct((NC, N_PAD, DH), jnp.float32),
            jax.ShapeDtypeStruct((N_PAD, 1), jnp.float32),
        ],
    )(xw, degp)


def _mid_body(z_ref, dinv_ref, b_ref, w_ref, y2_ref):
    dv = dinv_ref[...]
    z = jnp.concatenate([z_ref[0], z_ref[1]], axis=1)
    h = z * dv + b_ref[...]
    h = jnp.maximum(h, 0.0)
    y2 = jnp.dot(h, w_ref[...], preferred_element_type=jnp.float32) * dv
    y2_ref[...] = _split(y2)


def _tc_mid(z, dinv, b, w):
    return pl.pallas_call(
        _mid_body,
        grid=(_GRID,),
        in_specs=[
            pl.BlockSpec((NC, _BLK, DH), lambda i: (0, i, 0)),
            pl.BlockSpec((_BLK, 1), lambda i: (i, 0)),
            pl.BlockSpec((1, D), lambda i: (0, 0)),
            pl.BlockSpec((D, D), lambda i: (0, 0)),
        ],
        out_specs=pl.BlockSpec((NC, _BLK, DH), lambda i: (0, i, 0)),
        out_shape=jax.ShapeDtypeStruct((NC, N_PAD, DH), jnp.float32),
    )(z, dinv, b, w)


_FBLK = 2000
_FGRID = N // _FBLK


def _final_body(z_ref, dinv_ref, b_ref, o_ref):
    z = jnp.concatenate([z_ref[0], z_ref[1]], axis=1)
    o_ref[...] = z * dinv_ref[...] + b_ref[...]


def _tc_final(z, dinv, b):
    return pl.pallas_call(
        _final_body,
        grid=(_FGRID,),
        in_specs=[
            pl.BlockSpec((NC, _FBLK, DH), lambda i: (0, i, 0)),
            pl.BlockSpec((_FBLK, 1), lambda i: (i, 0)),
            pl.BlockSpec((1, D), lambda i: (0, 0)),
        ],
        out_specs=pl.BlockSpec((_FBLK, D), lambda i: (i, 0)),
        out_shape=jax.ShapeDtypeStruct((N, D), jnp.float32),
    )(z, dinv, b)


def kernel(x, edge_index, W1, b1, W2, b2):
    src = edge_index[0].astype(jnp.int32)
    dst = edge_index[1].astype(jnp.int32)
    pad_idx = N + (jnp.arange(E_PAD - E, dtype=jnp.int32) % PAD_ROWS)
    srcp = jnp.concatenate([src, pad_idx]).reshape(NW, CH, K)
    dstp = jnp.concatenate([dst, pad_idx]).reshape(NW, CH, K)
    x_pad = jnp.pad(x, ((0, N_PAD - N), (0, 0)))
    ones16 = jnp.ones((K, 16), jnp.float32)
    zeros16 = jnp.zeros((RPT, 16), jnp.float32)
    b1r = b1.reshape(1, D)
    b2r = b2.reshape(1, D)

    degp = _sc_degree(dstp, ones16, zeros16)
    xw1 = _tc_matmul(x_pad, W1)
    y1, dinv = _tc_scale(xw1, degp)
    z1 = _sc_aggregate(y1, srcp, dstp)
    y2 = _tc_mid(z1, dinv, b1r, W2)
    z2 = _sc_aggregate(y2, srcp, dstp)
    return _tc_final(z2, dinv, b2r)


# speedup vs baseline: 28.0170x; 1.0758x over previous
import dataclasses
import functools

import jax
import jax.numpy as jnp
from jax import lax
from jax.experimental import pallas as pl
from jax.experimental.pallas import tpu as pltpu
from jax.experimental.pallas import tpu_sc as plsc

N = 10000
E = 320000
D = 128

NC = 2
NS = 16
NW = NC * NS
K = 128
CH = 80
E_PAD = NW * CH * K
N_PAD = 10240
PAD_ROWS = N_PAD - N
RPT = N_PAD // NS

_mesh = plsc.VectorSubcoreMesh(core_axis_name="c", subcore_axis_name="s")

_sc_cp = pltpu.CompilerParams()
if "needs_layout_passes" in pltpu.CompilerParams.__dataclass_fields__:
    _sc_cp = dataclasses.replace(_sc_cp, needs_layout_passes=False)
_sc_linear_cp = pltpu.CompilerParams(use_tc_tiling_on_sc=False)


@functools.partial(
    pl.kernel,
    out_type=jax.ShapeDtypeStruct((NC, N_PAD, 16), jnp.float32),
    mesh=_mesh,
    scratch_types=[
        pltpu.VMEM((CH, K), jnp.int32),
        pltpu.VMEM((K, 16), jnp.float32),
        pltpu.VMEM_SHARED((N_PAD, 16), jnp.float32),
    ],
    compiler_params=_sc_linear_cp,
)
def _sc_degree(didx_hbm, ones_hbm, zeros16_hbm, out_hbm, didx_v, ones_v, dsh):
    c = lax.axis_index("c")
    s = lax.axis_index("s")
    w = c * NS + s
    base = s * RPT
    pltpu.sync_copy(didx_hbm.at[w], didx_v)
    pltpu.sync_copy(ones_hbm, ones_v)

    pltpu.sync_copy(zeros16_hbm, dsh.at[pl.ds(base, RPT)])
    plsc.subcore_barrier()

    @pl.loop(0, CH)
    def _(ci):
        pltpu.sync_copy(ones_v, dsh.at[didx_v.at[ci]], add=True)

    plsc.subcore_barrier()
    pltpu.sync_copy(dsh.at[pl.ds(base, RPT)],
                    out_hbm.at[c].at[pl.ds(base, RPT)])


DH = D // NC
CH2 = CH * 2


@functools.partial(
    pl.kernel,
    out_type=jax.ShapeDtypeStruct((NC, N_PAD, DH), jnp.float32),
    mesh=_mesh,
    scratch_types=[
        pltpu.VMEM((CH2, K), jnp.int32),
        pltpu.VMEM((CH2, K), jnp.int32),
        pltpu.VMEM((K, DH), jnp.float32),
        pltpu.VMEM((K, DH), jnp.float32),
        pltpu.VMEM((K, DH), jnp.float32),
        pltpu.VMEM((K, DH), jnp.float32),
        pltpu.VMEM_SHARED((N_PAD, DH), jnp.float32),
        pltpu.SemaphoreType.DMA,
        pltpu.SemaphoreType.DMA,
        pltpu.SemaphoreType.DMA,
        pltpu.SemaphoreType.DMA,
        pltpu.SemaphoreType.DMA,
        pltpu.SemaphoreType.DMA,
        pltpu.SemaphoreType.DMA,
        pltpu.SemaphoreType.DMA,
    ],
    compiler_params=_sc_linear_cp,
)
def _sc_aggregate(y_hbm, sidx_hbm, didx_hbm, out_hbm,
                  sidx_v, didx_v, g0, g1, g2, g3, zsh,
                  gs0, gs1, gs2, gs3, ss0, ss1, ss2, ss3):
    c = lax.axis_index("c")
    s = lax.axis_index("s")
    base = s * RPT
    yc = y_hbm.at[c]
    pltpu.sync_copy(sidx_hbm.at[2 * s], sidx_v.at[pl.ds(0, CH)])
    pltpu.sync_copy(sidx_hbm.at[2 * s + 1], sidx_v.at[pl.ds(CH, CH)])
    pltpu.sync_copy(didx_hbm.at[2 * s], didx_v.at[pl.ds(0, CH)])
    pltpu.sync_copy(didx_hbm.at[2 * s + 1], didx_v.at[pl.ds(CH, CH)])

    pltpu.sync_copy(yc.at[pl.ds(base, RPT)], zsh.at[pl.ds(base, RPT)])
    plsc.subcore_barrier()

    bufs = (g0, g1, g2, g3)
    gsems = (gs0, gs1, gs2, gs3)
    ssems = (ss0, ss1, ss2, ss3)

    for j in range(4):
        pltpu.make_async_copy(yc.at[sidx_v.at[j]], bufs[j], gsems[j]).start()

    @pl.loop(0, CH2, step=4)
    def _(ci):
        for j in range(4):
            cj = ci + j
            pltpu.make_async_copy(yc.at[sidx_v.at[cj]], bufs[j],
                                  gsems[j]).wait()
            pltpu.make_async_copy(bufs[j], zsh.at[didx_v.at[cj]],
                                  ssems[j]).start(add=True)

        for j in range(4):
            cj = ci + j + 4

            @pl.when(cj < CH2)
            def _():
                pltpu.make_async_copy(bufs[j], zsh.at[didx_v.at[0]],
                                      ssems[j]).wait()
                pltpu.make_async_copy(yc.at[sidx_v.at[cj]], bufs[j],
                                      gsems[j]).start()

    for j in range(4):
        pltpu.make_async_copy(bufs[j], zsh.at[didx_v.at[0]], ssems[j]).wait()

    plsc.subcore_barrier()
    pltpu.sync_copy(zsh.at[pl.ds(base, RPT)],
                    out_hbm.at[c].at[pl.ds(base, RPT)])


_BLK = 1024
_GRID = N_PAD // _BLK


def _mm_body(x_ref, w_ref, o_ref):
    o_ref[...] = jnp.dot(x_ref[...], w_ref[...],
                         preferred_element_type=jnp.float32)


def _tc_matmul(x, w):
    return pl.pallas_call(
        _mm_body,
        grid=(_GRID,),
        in_specs=[
            pl.BlockSpec((_BLK, D), lambda i: (i, 0)),
            pl.BlockSpec((D, D), lambda i: (0, 0)),
        ],
        out_specs=pl.BlockSpec((_BLK, D), lambda i: (i, 0)),
        out_shape=jax.ShapeDtypeStruct((N_PAD, D), jnp.float32),
    )(x, w)


def _split(t):
    return jnp.stack([t[:, :DH], t[:, DH:]], axis=0)


def _scale_body(xw_ref, deg_ref, y_ref, dinv_ref):
    deg = deg_ref[0, :, 0:1] + deg_ref[1, :, 0:1] + 1.0
    dv = lax.rsqrt(deg)
    dinv_ref[...] = dv
    y_ref[...] = _split(xw_ref[...] * dv)


def _tc_scale(xw, degp):
    return pl.pallas_call(
        _scale_body,
        grid=(_GRID,),
        in_specs=[
            pl.BlockSpec((_BLK, D), lambda i: (i, 0)),
            pl.BlockSpec((NC, _BLK, 16), lambda i: (0, i, 0)),
        ],
        out_specs=[
            pl.BlockSpec((NC, _BLK, DH), lambda i: (0, i, 0)),
            pl.BlockSpec((_BLK, 1), lambda i: (i, 0)),
        ],
        out_shape=[
            jax.ShapeDtypeStruct((NC, N_PAD, DH), jnp.float32),
            jax.ShapeDtypeStruct((N_PAD, 1), jnp.float32),
        ],
    )(xw, degp)


def _mid_body(z_ref, dinv_ref, b_ref, w_ref, y2_ref):
    dv = dinv_ref[...]
    z = jnp.concatenate([z_ref[0], z_ref[1]], axis=1)
    h = z * dv + b_ref[...]
    h = jnp.maximum(h, 0.0)
    y2 = jnp.dot(h, w_ref[...], preferred_element_type=jnp.float32) * dv
    y2_ref[...] = _split(y2)


def _tc_mid(z, dinv, b, w):
    return pl.pallas_call(
        _mid_body,
        grid=(_GRID,),
        in_specs=[
            pl.BlockSpec((NC, _BLK, DH), lambda i: (0, i, 0)),
            pl.BlockSpec((_BLK, 1), lambda i: (i, 0)),
            pl.BlockSpec((1, D), lambda i: (0, 0)),
            pl.BlockSpec((D, D), lambda i: (0, 0)),
        ],
        out_specs=pl.BlockSpec((NC, _BLK, DH), lambda i: (0, i, 0)),
        out_shape=jax.ShapeDtypeStruct((NC, N_PAD, DH), jnp.float32),
    )(z, dinv, b, w)


_FBLK = 2000
_FGRID = N // _FBLK


def _final_body(z_ref, dinv_ref, b_ref, o_ref):
    z = jnp.concatenate([z_ref[0], z_ref[1]], axis=1)
    o_ref[...] = z * dinv_ref[...] + b_ref[...]


def _tc_final(z, dinv, b):
    return pl.pallas_call(
        _final_body,
        grid=(_FGRID,),
        in_specs=[
            pl.BlockSpec((NC, _FBLK, DH), lambda i: (0, i, 0)),
            pl.BlockSpec((_FBLK, 1), lambda i: (i, 0)),
            pl.BlockSpec((1, D), lambda i: (0, 0)),
        ],
        out_specs=pl.BlockSpec((_FBLK, D), lambda i: (i, 0)),
        out_shape=jax.ShapeDtypeStruct((N, D), jnp.float32),
    )(z, dinv, b)


def kernel(x, edge_index, W1, b1, W2, b2):
    src = edge_index[0].astype(jnp.int32)
    dst = edge_index[1].astype(jnp.int32)
    pad_idx = N + (jnp.arange(E_PAD - E, dtype=jnp.int32) % PAD_ROWS)
    srcp = jnp.concatenate([src, pad_idx]).reshape(NW, CH, K)
    dstp = jnp.concatenate([dst, pad_idx]).reshape(NW, CH, K)
    x_pad = jnp.pad(x, ((0, N_PAD - N), (0, 0)))
    ones16 = jnp.ones((K, 16), jnp.float32)
    zeros16 = jnp.zeros((RPT, 16), jnp.float32)
    b1r = b1.reshape(1, D)
    b2r = b2.reshape(1, D)

    degp = _sc_degree(dstp, ones16, zeros16)
    xw1 = _tc_matmul(x_pad, W1)
    y1, dinv = _tc_scale(xw1, degp)
    z1 = _sc_aggregate(y1, srcp, dstp)
    y2 = _tc_mid(z1, dinv, b1r, W2)
    z2 = _sc_aggregate(y2, srcp, dstp)
    return _tc_final(z2, dinv, b2r)


# speedup vs baseline: 28.1558x; 1.0050x over previous
import dataclasses
import functools

import jax
import jax.numpy as jnp
from jax import lax
from jax.experimental import pallas as pl
from jax.experimental.pallas import tpu as pltpu
from jax.experimental.pallas import tpu_sc as plsc

N = 10000
E = 320000
D = 128

NC = 2
NS = 16
NW = NC * NS
K = 128
CH = 80
E_PAD = NW * CH * K
N_PAD = 10240
PAD_ROWS = N_PAD - N
RPT = N_PAD // NS

_mesh = plsc.VectorSubcoreMesh(core_axis_name="c", subcore_axis_name="s")

_sc_cp = pltpu.CompilerParams()
if "needs_layout_passes" in pltpu.CompilerParams.__dataclass_fields__:
    _sc_cp = dataclasses.replace(_sc_cp, needs_layout_passes=False)
_sc_linear_cp = pltpu.CompilerParams(use_tc_tiling_on_sc=False)


@functools.partial(
    pl.kernel,
    out_type=jax.ShapeDtypeStruct((NC, N_PAD, 16), jnp.float32),
    mesh=_mesh,
    scratch_types=[
        pltpu.VMEM((CH, K), jnp.int32),
        pltpu.VMEM((K, 16), jnp.float32),
        pltpu.VMEM_SHARED((N_PAD, 16), jnp.float32),
    ],
    compiler_params=_sc_linear_cp,
)
def _sc_degree(didx_hbm, ones_hbm, zeros16_hbm, out_hbm, didx_v, ones_v, dsh):
    c = lax.axis_index("c")
    s = lax.axis_index("s")
    w = c * NS + s
    base = s * RPT
    pltpu.sync_copy(didx_hbm.at[pl.ds(w * CH, CH)], didx_v)
    pltpu.sync_copy(ones_hbm, ones_v)

    pltpu.sync_copy(zeros16_hbm, dsh.at[pl.ds(base, RPT)])
    plsc.subcore_barrier()

    @pl.loop(0, CH)
    def _(ci):
        pltpu.sync_copy(ones_v, dsh.at[didx_v.at[ci]], add=True)

    plsc.subcore_barrier()
    pltpu.sync_copy(dsh.at[pl.ds(base, RPT)],
                    out_hbm.at[c].at[pl.ds(base, RPT)])


DH = D // NC
CH2 = CH * 2


@functools.partial(
    pl.kernel,
    out_type=jax.ShapeDtypeStruct((NC, N_PAD, DH), jnp.float32),
    mesh=_mesh,
    scratch_types=[
        pltpu.VMEM((CH2, K), jnp.int32),
        pltpu.VMEM((CH2, K), jnp.int32),
        pltpu.VMEM((K, DH), jnp.float32),
        pltpu.VMEM((K, DH), jnp.float32),
        pltpu.VMEM((K, DH), jnp.float32),
        pltpu.VMEM((K, DH), jnp.float32),
        pltpu.VMEM_SHARED((N_PAD, DH), jnp.float32),
        pltpu.SemaphoreType.DMA,
        pltpu.SemaphoreType.DMA,
        pltpu.SemaphoreType.DMA,
        pltpu.SemaphoreType.DMA,
        pltpu.SemaphoreType.DMA,
        pltpu.SemaphoreType.DMA,
        pltpu.SemaphoreType.DMA,
        pltpu.SemaphoreType.DMA,
    ],
    compiler_params=_sc_linear_cp,
)
def _sc_aggregate(y_hbm, sidx_hbm, didx_hbm, out_hbm,
                  sidx_v, didx_v, g0, g1, g2, g3, zsh,
                  gs0, gs1, gs2, gs3, ss0, ss1, ss2, ss3):
    c = lax.axis_index("c")
    s = lax.axis_index("s")
    base = s * RPT
    yc = y_hbm.at[c]
    pltpu.sync_copy(sidx_hbm.at[pl.ds(2 * s * CH, CH2)], sidx_v)
    pltpu.sync_copy(didx_hbm.at[pl.ds(2 * s * CH, CH2)], didx_v)

    pltpu.sync_copy(yc.at[pl.ds(base, RPT)], zsh.at[pl.ds(base, RPT)])
    plsc.subcore_barrier()

    bufs = (g0, g1, g2, g3)
    gsems = (gs0, gs1, gs2, gs3)
    ssems = (ss0, ss1, ss2, ss3)

    for j in range(4):
        pltpu.make_async_copy(yc.at[sidx_v.at[j]], bufs[j], gsems[j]).start()

    @pl.loop(0, CH2, step=4)
    def _(ci):
        for j in range(4):
            cj = ci + j
            pltpu.make_async_copy(yc.at[sidx_v.at[cj]], bufs[j],
                                  gsems[j]).wait()
            pltpu.make_async_copy(bufs[j], zsh.at[didx_v.at[cj]],
                                  ssems[j]).start(add=True)

        for j in range(4):
            cj = ci + j + 4

            @pl.when(cj < CH2)
            def _():
                pltpu.make_async_copy(bufs[j], zsh.at[didx_v.at[0]],
                                      ssems[j]).wait()
                pltpu.make_async_copy(yc.at[sidx_v.at[cj]], bufs[j],
                                      gsems[j]).start()

    for j in range(4):
        pltpu.make_async_copy(bufs[j], zsh.at[didx_v.at[0]], ssems[j]).wait()

    plsc.subcore_barrier()
    pltpu.sync_copy(zsh.at[pl.ds(base, RPT)],
                    out_hbm.at[c].at[pl.ds(base, RPT)])


_BLK = 1024
_GRID = N_PAD // _BLK


def _split(t):
    return jnp.stack([t[:, :DH], t[:, DH:]], axis=0)


def _mm_scale_body(x_ref, w_ref, deg_ref, y_ref, dinv_ref):
    deg = deg_ref[0, :, 0:1] + deg_ref[1, :, 0:1] + 1.0
    dv = lax.rsqrt(deg)
    dinv_ref[...] = dv
    xw = jnp.dot(x_ref[...], w_ref[...], preferred_element_type=jnp.float32)
    y_ref[...] = _split(xw * dv)


def _tc_mm_scale(x, w, degp):
    return pl.pallas_call(
        _mm_scale_body,
        grid=(_GRID,),
        in_specs=[
            pl.BlockSpec((_BLK, D), lambda i: (i, 0)),
            pl.BlockSpec((D, D), lambda i: (0, 0)),
            pl.BlockSpec((NC, _BLK, 16), lambda i: (0, i, 0)),
        ],
        out_specs=[
            pl.BlockSpec((NC, _BLK, DH), lambda i: (0, i, 0)),
            pl.BlockSpec((_BLK, 1), lambda i: (i, 0)),
        ],
        out_shape=[
            jax.ShapeDtypeStruct((NC, N_PAD, DH), jnp.float32),
            jax.ShapeDtypeStruct((N_PAD, 1), jnp.float32),
        ],
    )(x, w, degp)


def _mid_body(z_ref, dinv_ref, b_ref, w_ref, y2_ref):
    dv = dinv_ref[...]
    z = jnp.concatenate([z_ref[0], z_ref[1]], axis=1)
    h = z * dv + b_ref[...]
    h = jnp.maximum(h, 0.0)
    y2 = jnp.dot(h, w_ref[...], preferred_element_type=jnp.float32) * dv
    y2_ref[...] = _split(y2)


def _tc_mid(z, dinv, b, w):
    return pl.pallas_call(
        _mid_body,
        grid=(_GRID,),
        in_specs=[
            pl.BlockSpec((NC, _BLK, DH), lambda i: (0, i, 0)),
            pl.BlockSpec((_BLK, 1), lambda i: (i, 0)),
            pl.BlockSpec((1, D), lambda i: (0, 0)),
            pl.BlockSpec((D, D), lambda i: (0, 0)),
        ],
        out_specs=pl.BlockSpec((NC, _BLK, DH), lambda i: (0, i, 0)),
        out_shape=jax.ShapeDtypeStruct((NC, N_PAD, DH), jnp.float32),
    )(z, dinv, b, w)


_FBLK = 2000
_FGRID = N // _FBLK


def _final_body(z_ref, dinv_ref, b_ref, o_ref):
    z = jnp.concatenate([z_ref[0], z_ref[1]], axis=1)
    o_ref[...] = z * dinv_ref[...] + b_ref[...]


def _tc_final(z, dinv, b):
    return pl.pallas_call(
        _final_body,
        grid=(_FGRID,),
        in_specs=[
            pl.BlockSpec((NC, _FBLK, DH), lambda i: (0, i, 0)),
            pl.BlockSpec((_FBLK, 1), lambda i: (i, 0)),
            pl.BlockSpec((1, D), lambda i: (0, 0)),
        ],
        out_specs=pl.BlockSpec((_FBLK, D), lambda i: (i, 0)),
        out_shape=jax.ShapeDtypeStruct((N, D), jnp.float32),
    )(z, dinv, b)


def kernel(x, edge_index, W1, b1, W2, b2):
    src = edge_index[0].astype(jnp.int32)
    dst = edge_index[1].astype(jnp.int32)
    pad_idx = N + (jnp.arange(E_PAD - E, dtype=jnp.int32) % PAD_ROWS)
    srcp = jnp.concatenate([src, pad_idx]).reshape(NW * CH, K)
    dstp = jnp.concatenate([dst, pad_idx]).reshape(NW * CH, K)
    x_pad = jnp.pad(x, ((0, N_PAD - N), (0, 0)))
    ones16 = jnp.ones((K, 16), jnp.float32)
    zeros16 = jnp.zeros((RPT, 16), jnp.float32)
    b1r = b1.reshape(1, D)
    b2r = b2.reshape(1, D)

    degp = _sc_degree(dstp, ones16, zeros16)
    y1, dinv = _tc_mm_scale(x_pad, W1, degp)
    z1 = _sc_aggregate(y1, srcp, dstp)
    y2 = _tc_mid(z1, dinv, b1r, W2)
    z2 = _sc_aggregate(y2, srcp, dstp)
    return _tc_final(z2, dinv, b2r)


# speedup vs baseline: 31.2885x; 1.1113x over previous
import dataclasses
import functools

import jax
import jax.numpy as jnp
from jax import lax
from jax.experimental import pallas as pl
from jax.experimental.pallas import tpu as pltpu
from jax.experimental.pallas import tpu_sc as plsc

N = 10000
E = 320000
D = 128

NC = 2
NS = 16
NW = NC * NS
K = 128
CH = 80
E_PAD = NW * CH * K
N_PAD = 10240
PAD_ROWS = N_PAD - N
RPT = N_PAD // NS

_mesh = plsc.VectorSubcoreMesh(core_axis_name="c", subcore_axis_name="s")

_sc_cp = pltpu.CompilerParams()
if "needs_layout_passes" in pltpu.CompilerParams.__dataclass_fields__:
    _sc_cp = dataclasses.replace(_sc_cp, needs_layout_passes=False)
_sc_linear_cp = pltpu.CompilerParams(use_tc_tiling_on_sc=False)


@functools.partial(
    pl.kernel,
    out_type=jax.ShapeDtypeStruct((NC, N_PAD, 16), jnp.float32),
    mesh=_mesh,
    scratch_types=[
        pltpu.VMEM((CH, K), jnp.int32),
        pltpu.VMEM((K, 16), jnp.float32),
        pltpu.VMEM_SHARED((N_PAD, 16), jnp.float32),
    ],
    compiler_params=_sc_linear_cp,
)
def _sc_degree(didx_hbm, ones_hbm, zeros16_hbm, out_hbm, didx_v, ones_v, dsh):
    c = lax.axis_index("c")
    s = lax.axis_index("s")
    w = c * NS + s
    base = s * RPT
    pltpu.sync_copy(didx_hbm.at[pl.ds(w * CH, CH)], didx_v)
    pltpu.sync_copy(ones_hbm, ones_v)

    pltpu.sync_copy(zeros16_hbm, dsh.at[pl.ds(base, RPT)])
    plsc.subcore_barrier()

    @pl.loop(0, CH)
    def _(ci):
        pltpu.sync_copy(ones_v, dsh.at[didx_v.at[ci]], add=True)

    plsc.subcore_barrier()
    pltpu.sync_copy(dsh.at[pl.ds(base, RPT)],
                    out_hbm.at[c].at[pl.ds(base, RPT)])


DH = D // NC
CH2 = CH * 2
NP2 = N_PAD // 2


@functools.partial(
    pl.kernel,
    out_type=jax.ShapeDtypeStruct((NC, N_PAD, DH), jnp.float32),
    mesh=_mesh,
    scratch_types=[
        pltpu.VMEM((CH2, K), jnp.int32),
        pltpu.VMEM((CH2, K), jnp.int32),
        pltpu.VMEM((K, DH), jnp.float32),
        pltpu.VMEM((K, DH), jnp.float32),
        pltpu.VMEM((K, DH), jnp.float32),
        pltpu.VMEM((K, DH), jnp.float32),
        pltpu.VMEM_SHARED((N_PAD, DH), jnp.float32),
        pltpu.SemaphoreType.DMA,
        pltpu.SemaphoreType.DMA,
        pltpu.SemaphoreType.DMA,
        pltpu.SemaphoreType.DMA,
        pltpu.SemaphoreType.DMA,
        pltpu.SemaphoreType.DMA,
        pltpu.SemaphoreType.DMA,
        pltpu.SemaphoreType.DMA,
    ],
    compiler_params=_sc_linear_cp,
)
def _sc_aggregate(y_hbm, sidx_hbm, didx_hbm, out_hbm,
                  sidx_v, didx_v, g0, g1, g2, g3, zsh,
                  gs0, gs1, gs2, gs3, ss0, ss1, ss2, ss3):
    c = lax.axis_index("c")
    s = lax.axis_index("s")
    base = s * RPT
    yc = y_hbm.at[c]
    oc = out_hbm.at[c]
    pltpu.sync_copy(sidx_hbm.at[pl.ds(2 * s * CH, CH2)], sidx_v)
    pltpu.sync_copy(didx_hbm.at[pl.ds(2 * s * CH, CH2)], didx_v)

    pltpu.sync_copy(yc.at[pl.ds(base, RPT)], zsh.at[pl.ds(base, RPT)])
    plsc.subcore_barrier()

    bufs = (g0, g1, g2, g3)
    gsems = (gs0, gs1, gs2, gs3)
    ssems = (ss0, ss1, ss2, ss3)

    for j in range(4):
        pltpu.make_async_copy(yc.at[sidx_v.at[j]], bufs[j], gsems[j]).start()

    @pl.loop(0, CH2, step=4)
    def _(ci):
        for j in range(4):
            cj = ci + j
            pltpu.make_async_copy(yc.at[sidx_v.at[cj]], bufs[j],
                                  gsems[j]).wait()
            pltpu.make_async_copy(bufs[j], zsh.at[didx_v.at[cj]],
                                  ssems[j]).start(add=True)

        for j in range(4):
            cj = ci + j + 4

            @pl.when(cj < CH2)
            def _():
                pltpu.make_async_copy(bufs[j], zsh.at[didx_v.at[0]],
                                      ssems[j]).wait()
                pltpu.make_async_copy(yc.at[sidx_v.at[cj]], bufs[j],
                                      gsems[j]).start()

    for j in range(4):
        pltpu.make_async_copy(bufs[j], zsh.at[didx_v.at[0]], ssems[j]).wait()

    plsc.subcore_barrier()
    pltpu.sync_copy(zsh.at[pl.ds(base, RPT)], oc.at[pl.ds(base, RPT)])


_BLK = 1024
_GRID = N_PAD // _BLK


_HB = _BLK // 2


def _split(t):
    return jnp.stack(
        [jnp.concatenate([t[:_HB, :DH], t[_HB:, :DH]], axis=1),
         jnp.concatenate([t[:_HB, DH:], t[_HB:, DH:]], axis=1)], axis=0)


def _unsplit(z0, z1):
    return jnp.concatenate(
        [jnp.concatenate([z0[:, :DH], z1[:, :DH]], axis=1),
         jnp.concatenate([z0[:, DH:], z1[:, DH:]], axis=1)], axis=0)


def _mm_scale_body(x_ref, w_ref, deg_ref, y_ref, dinv_ref):
    deg = deg_ref[0, :, 0:1] + deg_ref[1, :, 0:1] + 1.0
    dv = lax.rsqrt(deg)
    dinv_ref[...] = dv
    xw = jnp.dot(x_ref[...], w_ref[...], preferred_element_type=jnp.float32)
    y_ref[...] = _split(xw * dv)


def _tc_mm_scale(x, w, degp):
    return pl.pallas_call(
        _mm_scale_body,
        grid=(_GRID,),
        in_specs=[
            pl.BlockSpec((_BLK, D), lambda i: (i, 0)),
            pl.BlockSpec((D, D), lambda i: (0, 0)),
            pl.BlockSpec((NC, _BLK, 16), lambda i: (0, i, 0)),
        ],
        out_specs=[
            pl.BlockSpec((NC, _BLK // 2, D), lambda i: (0, i, 0)),
            pl.BlockSpec((_BLK, 1), lambda i: (i, 0)),
        ],
        out_shape=[
            jax.ShapeDtypeStruct((NC, NP2, D), jnp.float32),
            jax.ShapeDtypeStruct((N_PAD, 1), jnp.float32),
        ],
    )(x, w, degp)


def _mid_body(z_ref, dinv_ref, b_ref, w_ref, y2_ref):
    dv = dinv_ref[...]
    z = _unsplit(z_ref[0], z_ref[1])
    h = z * dv + b_ref[...]
    h = jnp.maximum(h, 0.0)
    y2 = jnp.dot(h, w_ref[...], preferred_element_type=jnp.float32) * dv
    y2_ref[...] = _split(y2)


def _tc_mid(z, dinv, b, w):
    return pl.pallas_call(
        _mid_body,
        grid=(_GRID,),
        in_specs=[
            pl.BlockSpec((NC, _BLK // 2, D), lambda i: (0, i, 0)),
            pl.BlockSpec((_BLK, 1), lambda i: (i, 0)),
            pl.BlockSpec((1, D), lambda i: (0, 0)),
            pl.BlockSpec((D, D), lambda i: (0, 0)),
        ],
        out_specs=pl.BlockSpec((NC, _BLK // 2, D), lambda i: (0, i, 0)),
        out_shape=jax.ShapeDtypeStruct((NC, NP2, D), jnp.float32),
    )(z, dinv, b, w)


def _final_body(z_ref, dinv_ref, b_ref, o_ref):
    z = _unsplit(z_ref[0], z_ref[1])
    o_ref[...] = z * dinv_ref[...] + b_ref[...]


def _tc_final(z, dinv, b):
    return pl.pallas_call(
        _final_body,
        grid=(_GRID,),
        in_specs=[
            pl.BlockSpec((NC, _HB, D), lambda i: (0, i, 0)),
            pl.BlockSpec((_BLK, 1), lambda i: (i, 0)),
            pl.BlockSpec((1, D), lambda i: (0, 0)),
        ],
        out_specs=pl.BlockSpec((_BLK, D), lambda i: (i, 0)),
        out_shape=jax.ShapeDtypeStruct((N_PAD, D), jnp.float32),
    )(z, dinv, b)


def kernel(x, edge_index, W1, b1, W2, b2):
    src = edge_index[0].astype(jnp.int32)
    dst = edge_index[1].astype(jnp.int32)
    pad_idx = N + (jnp.arange(E_PAD - E, dtype=jnp.int32) % PAD_ROWS)

    def _perm(idx):
        l = idx % _BLK
        return (idx - l) + jnp.where(l < _HB, 2 * l, 2 * (l - _HB) + 1)

    srcp = _perm(jnp.concatenate([src, pad_idx])).reshape(NW * CH, K)
    dstp = _perm(jnp.concatenate([dst, pad_idx])).reshape(NW * CH, K)
    dst_deg = jnp.concatenate([dst, pad_idx]).reshape(NW * CH, K)
    x_pad = jnp.pad(x, ((0, N_PAD - N), (0, 0)))
    ones16 = jnp.ones((K, 16), jnp.float32)
    zeros16 = jnp.zeros((RPT, 16), jnp.float32)
    b1r = b1.reshape(1, D)
    b2r = b2.reshape(1, D)

    degp = _sc_degree(dst_deg, ones16, zeros16)
    y1, dinv = _tc_mm_scale(x_pad, W1, degp)
    z1 = _sc_aggregate(y1.reshape(NC, N_PAD, DH), srcp, dstp)
    y2 = _tc_mid(z1.reshape(NC, NP2, D), dinv, b1r, W2)
    z2 = _sc_aggregate(y2.reshape(NC, N_PAD, DH), srcp, dstp)
    return _tc_final(z2.reshape(NC, NP2, D), dinv, b2r)[:N]


# speedup vs baseline: 31.7017x; 1.0132x over previous
import dataclasses
import functools

import jax
import jax.numpy as jnp
from jax import lax
from jax.experimental import pallas as pl
from jax.experimental.pallas import tpu as pltpu
from jax.experimental.pallas import tpu_sc as plsc

N = 10000
E = 320000
D = 128

NC = 2
NS = 16
NW = NC * NS
K = 128
CH = 80
E_PAD = NW * CH * K
N_PAD = 10240
PAD_ROWS = N_PAD - N
RPT = N_PAD // NS

_mesh = plsc.VectorSubcoreMesh(core_axis_name="c", subcore_axis_name="s")

_sc_cp = pltpu.CompilerParams()
if "needs_layout_passes" in pltpu.CompilerParams.__dataclass_fields__:
    _sc_cp = dataclasses.replace(_sc_cp, needs_layout_passes=False)
_sc_linear_cp = pltpu.CompilerParams(use_tc_tiling_on_sc=False)


@functools.partial(
    pl.kernel,
    out_type=jax.ShapeDtypeStruct((NC, N_PAD, 16), jnp.float32),
    mesh=_mesh,
    scratch_types=[
        pltpu.VMEM((CH, K), jnp.int32),
        pltpu.VMEM((K, 16), jnp.float32),
        pltpu.VMEM_SHARED((N_PAD, 16), jnp.float32),
    ],
    compiler_params=_sc_linear_cp,
)
def _sc_degree(didx_hbm, ones_hbm, zeros16_hbm, out_hbm, didx_v, ones_v, dsh):
    c = lax.axis_index("c")
    s = lax.axis_index("s")
    w = c * NS + s
    base = s * RPT
    pltpu.sync_copy(didx_hbm.at[pl.ds(w * CH, CH)], didx_v)
    pltpu.sync_copy(ones_hbm, ones_v)

    pltpu.sync_copy(zeros16_hbm, dsh.at[pl.ds(base, RPT)])
    plsc.subcore_barrier()

    @pl.loop(0, CH)
    def _(ci):
        pltpu.sync_copy(ones_v, dsh.at[didx_v.at[ci]], add=True)

    plsc.subcore_barrier()
    pltpu.sync_copy(dsh.at[pl.ds(base, RPT)],
                    out_hbm.at[c].at[pl.ds(base, RPT)])


DH = D // NC
CH2 = CH * 2
NP2 = N_PAD // 2


@functools.partial(
    pl.kernel,
    out_type=jax.ShapeDtypeStruct((NC, N_PAD, DH), jnp.float32),
    mesh=_mesh,
    scratch_types=[
        pltpu.VMEM((CH2, K), jnp.int32),
        pltpu.VMEM((CH2, K), jnp.int32),
        pltpu.VMEM((K, DH), jnp.float32),
        pltpu.VMEM((K, DH), jnp.float32),
        pltpu.VMEM((K, DH), jnp.float32),
        pltpu.VMEM((K, DH), jnp.float32),
        pltpu.VMEM_SHARED((N_PAD, DH), jnp.float32),
        pltpu.SemaphoreType.DMA,
        pltpu.SemaphoreType.DMA,
        pltpu.SemaphoreType.DMA,
        pltpu.SemaphoreType.DMA,
        pltpu.SemaphoreType.DMA,
        pltpu.SemaphoreType.DMA,
        pltpu.SemaphoreType.DMA,
        pltpu.SemaphoreType.DMA,
    ],
    compiler_params=_sc_linear_cp,
)
def _sc_aggregate(y_hbm, sidx_hbm, didx_hbm, out_hbm,
                  sidx_v, didx_v, g0, g1, g2, g3, zsh,
                  gs0, gs1, gs2, gs3, ss0, ss1, ss2, ss3):
    c = lax.axis_index("c")
    s = lax.axis_index("s")
    base = s * RPT
    yc = y_hbm.at[c]
    oc = out_hbm.at[c]
    pltpu.sync_copy(sidx_hbm.at[pl.ds(2 * s * CH, CH2)], sidx_v)
    pltpu.sync_copy(didx_hbm.at[pl.ds(2 * s * CH, CH2)], didx_v)

    pltpu.sync_copy(yc.at[pl.ds(base, RPT)], zsh.at[pl.ds(base, RPT)])
    plsc.subcore_barrier()

    bufs = (g0, g1, g2, g3)
    gsems = (gs0, gs1, gs2, gs3)
    ssems = (ss0, ss1, ss2, ss3)

    for j in range(4):
        pltpu.make_async_copy(yc.at[sidx_v.at[j]], bufs[j], gsems[j]).start()

    @pl.loop(0, CH2, step=4)
    def _(ci):
        for j in range(4):
            cj = ci + j
            pltpu.make_async_copy(yc.at[sidx_v.at[cj]], bufs[j],
                                  gsems[j]).wait()
            pltpu.make_async_copy(bufs[j], zsh.at[didx_v.at[cj]],
                                  ssems[j]).start(add=True)

        for j in range(4):
            cj = ci + j + 4

            @pl.when(cj < CH2)
            def _():
                pltpu.make_async_copy(bufs[j], zsh.at[didx_v.at[0]],
                                      ssems[j]).wait()
                pltpu.make_async_copy(yc.at[sidx_v.at[cj]], bufs[j],
                                      gsems[j]).start()

    for j in range(4):
        pltpu.make_async_copy(bufs[j], zsh.at[didx_v.at[0]], ssems[j]).wait()

    plsc.subcore_barrier()
    pltpu.sync_copy(zsh.at[pl.ds(base, RPT)], oc.at[pl.ds(base, RPT)])


_BLK = 2048
_GRID = N_PAD // _BLK


_HB = _BLK // 2


def _split(t):
    return jnp.stack(
        [jnp.concatenate([t[:_HB, :DH], t[_HB:, :DH]], axis=1),
         jnp.concatenate([t[:_HB, DH:], t[_HB:, DH:]], axis=1)], axis=0)


def _unsplit(z0, z1):
    return jnp.concatenate(
        [jnp.concatenate([z0[:, :DH], z1[:, :DH]], axis=1),
         jnp.concatenate([z0[:, DH:], z1[:, DH:]], axis=1)], axis=0)


def _mm_scale_body(x_ref, w_ref, deg_ref, y_ref, dinv_ref):
    deg = deg_ref[0, :, 0:1] + deg_ref[1, :, 0:1] + 1.0
    dv = lax.rsqrt(deg)
    dinv_ref[...] = dv
    xw = jnp.dot(x_ref[...], w_ref[...], preferred_element_type=jnp.float32)
    y_ref[...] = _split(xw * dv)


def _tc_mm_scale(x, w, degp):
    return pl.pallas_call(
        _mm_scale_body,
        grid=(_GRID,),
        in_specs=[
            pl.BlockSpec((_BLK, D), lambda i: (i, 0)),
            pl.BlockSpec((D, D), lambda i: (0, 0)),
            pl.BlockSpec((NC, _BLK, 16), lambda i: (0, i, 0)),
        ],
        out_specs=[
            pl.BlockSpec((NC, _BLK // 2, D), lambda i: (0, i, 0)),
            pl.BlockSpec((_BLK, 1), lambda i: (i, 0)),
        ],
        out_shape=[
            jax.ShapeDtypeStruct((NC, NP2, D), jnp.float32),
            jax.ShapeDtypeStruct((N_PAD, 1), jnp.float32),
        ],
    )(x, w, degp)


def _mid_body(z_ref, dinv_ref, b_ref, w_ref, y2_ref):
    dv = dinv_ref[...]
    z = _unsplit(z_ref[0], z_ref[1])
    h = z * dv + b_ref[...]
    h = jnp.maximum(h, 0.0)
    y2 = jnp.dot(h, w_ref[...], preferred_element_type=jnp.float32) * dv
    y2_ref[...] = _split(y2)


def _tc_mid(z, dinv, b, w):
    return pl.pallas_call(
        _mid_body,
        grid=(_GRID,),
        in_specs=[
            pl.BlockSpec((NC, _BLK // 2, D), lambda i: (0, i, 0)),
            pl.BlockSpec((_BLK, 1), lambda i: (i, 0)),
            pl.BlockSpec((1, D), lambda i: (0, 0)),
            pl.BlockSpec((D, D), lambda i: (0, 0)),
        ],
        out_specs=pl.BlockSpec((NC, _BLK // 2, D), lambda i: (0, i, 0)),
        out_shape=jax.ShapeDtypeStruct((NC, NP2, D), jnp.float32),
    )(z, dinv, b, w)


def _final_body(z_ref, dinv_ref, b_ref, o_ref):
    z = _unsplit(z_ref[0], z_ref[1])
    o_ref[...] = z * dinv_ref[...] + b_ref[...]


def _tc_final(z, dinv, b):
    return pl.pallas_call(
        _final_body,
        grid=(_GRID,),
        in_specs=[
            pl.BlockSpec((NC, _HB, D), lambda i: (0, i, 0)),
            pl.BlockSpec((_BLK, 1), lambda i: (i, 0)),
            pl.BlockSpec((1, D), lambda i: (0, 0)),
        ],
        out_specs=pl.BlockSpec((_BLK, D), lambda i: (i, 0)),
        out_shape=jax.ShapeDtypeStruct((N_PAD, D), jnp.float32),
    )(z, dinv, b)


def kernel(x, edge_index, W1, b1, W2, b2):
    src = edge_index[0].astype(jnp.int32)
    dst = edge_index[1].astype(jnp.int32)
    pad_idx = N + (jnp.arange(E_PAD - E, dtype=jnp.int32) % PAD_ROWS)

    def _perm(idx):
        l = idx % _BLK
        return (idx - l) + jnp.where(l < _HB, 2 * l, 2 * (l - _HB) + 1)

    srcp = _perm(jnp.concatenate([src, pad_idx])).reshape(NW * CH, K)
    dstp = _perm(jnp.concatenate([dst, pad_idx])).reshape(NW * CH, K)
    dst_deg = jnp.concatenate([dst, pad_idx]).reshape(NW * CH, K)
    x_pad = jnp.pad(x, ((0, N_PAD - N), (0, 0)))
    ones16 = jnp.ones((K, 16), jnp.float32)
    zeros16 = jnp.zeros((RPT, 16), jnp.float32)
    b1r = b1.reshape(1, D)
    b2r = b2.reshape(1, D)

    degp = _sc_degree(dst_deg, ones16, zeros16)
    y1, dinv = _tc_mm_scale(x_pad, W1, degp)
    z1 = _sc_aggregate(y1.reshape(NC, N_PAD, DH), srcp, dstp)
    y2 = _tc_mid(z1.reshape(NC, NP2, D), dinv, b1r, W2)
    z2 = _sc_aggregate(y2.reshape(NC, N_PAD, DH), srcp, dstp)
    return _tc_final(z2.reshape(NC, NP2, D), dinv, b2r)[:N]


# speedup vs baseline: 32.1890x; 1.0154x over previous
import dataclasses
import functools

import jax
import jax.numpy as jnp
from jax import lax
from jax.experimental import pallas as pl
from jax.experimental.pallas import tpu as pltpu
from jax.experimental.pallas import tpu_sc as plsc

N = 10000
E = 320000
D = 128

NC = 2
NS = 16
NW = NC * NS
K = 128
CH = 80
E_PAD = NW * CH * K
N_PAD = 10240
PAD_ROWS = N_PAD - N
RPT = N_PAD // NS

_mesh = plsc.VectorSubcoreMesh(core_axis_name="c", subcore_axis_name="s")

_sc_cp = pltpu.CompilerParams()
if "needs_layout_passes" in pltpu.CompilerParams.__dataclass_fields__:
    _sc_cp = dataclasses.replace(_sc_cp, needs_layout_passes=False)
_sc_linear_cp = pltpu.CompilerParams(use_tc_tiling_on_sc=False)


DCH = E // K // NW
DEXTRA = E // K - DCH * NW


@functools.partial(
    pl.kernel,
    out_type=jax.ShapeDtypeStruct((NC, N_PAD, 16), jnp.float32),
    mesh=_mesh,
    scratch_types=[
        pltpu.VMEM((DCH + 1, K), jnp.int32),
        pltpu.VMEM((K, 16), jnp.float32),
        pltpu.VMEM_SHARED((N_PAD, 16), jnp.float32),
    ],
    compiler_params=_sc_linear_cp,
)
def _sc_degree(didx_hbm, ones_hbm, zeros16_hbm, out_hbm, didx_v, ones_v, dsh):
    c = lax.axis_index("c")
    s = lax.axis_index("s")
    w = c * NS + s
    base = s * RPT
    pltpu.sync_copy(didx_hbm.at[pl.ds(w * DCH, DCH)],
                    didx_v.at[pl.ds(0, DCH)])

    @pl.when(w < DEXTRA)
    def _():
        pltpu.sync_copy(didx_hbm.at[pl.ds(NW * DCH + w, 1)],
                        didx_v.at[pl.ds(DCH, 1)])

    pltpu.sync_copy(ones_hbm, ones_v)
    pltpu.sync_copy(zeros16_hbm, dsh.at[pl.ds(base, RPT)])
    plsc.subcore_barrier()

    @pl.loop(0, DCH)
    def _(ci):
        pltpu.sync_copy(ones_v, dsh.at[didx_v.at[ci]], add=True)

    @pl.when(w < DEXTRA)
    def _():
        pltpu.sync_copy(ones_v, dsh.at[didx_v.at[DCH]], add=True)

    plsc.subcore_barrier()
    pltpu.sync_copy(dsh.at[pl.ds(base, RPT)],
                    out_hbm.at[c].at[pl.ds(base, RPT)])


DH = D // NC
CH2 = CH * 2
NP2 = N_PAD // 2


@functools.partial(
    pl.kernel,
    out_type=jax.ShapeDtypeStruct((NC, N_PAD, DH), jnp.float32),
    mesh=_mesh,
    scratch_types=[
        pltpu.VMEM((CH2, K), jnp.int32),
        pltpu.VMEM((CH2, K), jnp.int32),
        pltpu.VMEM((K, DH), jnp.float32),
        pltpu.VMEM((K, DH), jnp.float32),
        pltpu.VMEM((K, DH), jnp.float32),
        pltpu.VMEM((K, DH), jnp.float32),
        pltpu.VMEM_SHARED((N_PAD, DH), jnp.float32),
        pltpu.SemaphoreType.DMA,
        pltpu.SemaphoreType.DMA,
        pltpu.SemaphoreType.DMA,
        pltpu.SemaphoreType.DMA,
        pltpu.SemaphoreType.DMA,
        pltpu.SemaphoreType.DMA,
        pltpu.SemaphoreType.DMA,
        pltpu.SemaphoreType.DMA,
    ],
    compiler_params=_sc_linear_cp,
)
def _sc_aggregate(y_hbm, sidx_hbm, didx_hbm, out_hbm,
                  sidx_v, didx_v, g0, g1, g2, g3, zsh,
                  gs0, gs1, gs2, gs3, ss0, ss1, ss2, ss3):
    c = lax.axis_index("c")
    s = lax.axis_index("s")
    base = s * RPT
    yc = y_hbm.at[c]
    oc = out_hbm.at[c]
    pltpu.sync_copy(sidx_hbm.at[pl.ds(2 * s * CH, CH2)], sidx_v)
    pltpu.sync_copy(didx_hbm.at[pl.ds(2 * s * CH, CH2)], didx_v)

    pltpu.sync_copy(yc.at[pl.ds(base, RPT)], zsh.at[pl.ds(base, RPT)])
    plsc.subcore_barrier()

    bufs = (g0, g1, g2, g3)
    gsems = (gs0, gs1, gs2, gs3)
    ssems = (ss0, ss1, ss2, ss3)

    for j in range(4):
        pltpu.make_async_copy(yc.at[sidx_v.at[j]], bufs[j], gsems[j]).start()

    @pl.loop(0, CH2, step=4)
    def _(ci):
        for j in range(4):
            cj = ci + j
            pltpu.make_async_copy(yc.at[sidx_v.at[cj]], bufs[j],
                                  gsems[j]).wait()
            pltpu.make_async_copy(bufs[j], zsh.at[didx_v.at[cj]],
                                  ssems[j]).start(add=True)

        for j in range(4):
            cj = ci + j + 4

            @pl.when(cj < CH2)
            def _():
                pltpu.make_async_copy(bufs[j], zsh.at[didx_v.at[0]],
                                      ssems[j]).wait()
                pltpu.make_async_copy(yc.at[sidx_v.at[cj]], bufs[j],
                                      gsems[j]).start()

    for j in range(4):
        pltpu.make_async_copy(bufs[j], zsh.at[didx_v.at[0]], ssems[j]).wait()

    plsc.subcore_barrier()
    pltpu.sync_copy(zsh.at[pl.ds(base, RPT)], oc.at[pl.ds(base, RPT)])


_BLK = 2048
_GRID = N_PAD // _BLK


_HB = _BLK // 2


def _split(t):
    return jnp.stack(
        [jnp.concatenate([t[:_HB, :DH], t[_HB:, :DH]], axis=1),
         jnp.concatenate([t[:_HB, DH:], t[_HB:, DH:]], axis=1)], axis=0)


def _unsplit(z0, z1):
    return jnp.concatenate(
        [jnp.concatenate([z0[:, :DH], z1[:, :DH]], axis=1),
         jnp.concatenate([z0[:, DH:], z1[:, DH:]], axis=1)], axis=0)


def _mm_scale_body(x_ref, w_ref, deg_ref, y_ref, dinv_ref):
    deg = deg_ref[0, :, 0:1] + deg_ref[1, :, 0:1] + 1.0
    dv = lax.rsqrt(deg)
    dinv_ref[...] = dv
    xw = jnp.dot(x_ref[...], w_ref[...], preferred_element_type=jnp.float32)
    y_ref[...] = _split(xw * dv)


def _tc_mm_scale(x, w, degp):
    return pl.pallas_call(
        _mm_scale_body,
        grid=(_GRID,),
        in_specs=[
            pl.BlockSpec((_BLK, D), lambda i: (i, 0)),
            pl.BlockSpec((D, D), lambda i: (0, 0)),
            pl.BlockSpec((NC, _BLK, 16), lambda i: (0, i, 0)),
        ],
        out_specs=[
            pl.BlockSpec((NC, _BLK // 2, D), lambda i: (0, i, 0)),
            pl.BlockSpec((_BLK, 1), lambda i: (i, 0)),
        ],
        out_shape=[
            jax.ShapeDtypeStruct((NC, NP2, D), jnp.float32),
            jax.ShapeDtypeStruct((N_PAD, 1), jnp.float32),
        ],
    )(x, w, degp)


def _mid_body(z_ref, dinv_ref, b_ref, w_ref, y2_ref):
    dv = dinv_ref[...]
    z = _unsplit(z_ref[0], z_ref[1])
    h = z * dv + b_ref[...]
    h = jnp.maximum(h, 0.0)
    y2 = jnp.dot(h, w_ref[...], preferred_element_type=jnp.float32) * dv
    y2_ref[...] = _split(y2)


def _tc_mid(z, dinv, b, w):
    return pl.pallas_call(
        _mid_body,
        grid=(_GRID,),
        in_specs=[
            pl.BlockSpec((NC, _BLK // 2, D), lambda i: (0, i, 0)),
            pl.BlockSpec((_BLK, 1), lambda i: (i, 0)),
            pl.BlockSpec((1, D), lambda i: (0, 0)),
            pl.BlockSpec((D, D), lambda i: (0, 0)),
        ],
        out_specs=pl.BlockSpec((NC, _BLK // 2, D), lambda i: (0, i, 0)),
        out_shape=jax.ShapeDtypeStruct((NC, NP2, D), jnp.float32),
    )(z, dinv, b, w)


def _final_body(z_ref, dinv_ref, b_ref, o_ref):
    z = _unsplit(z_ref[0], z_ref[1])
    o_ref[...] = z * dinv_ref[...] + b_ref[...]


def _tc_final(z, dinv, b):
    return pl.pallas_call(
        _final_body,
        grid=(_GRID,),
        in_specs=[
            pl.BlockSpec((NC, _HB, D), lambda i: (0, i, 0)),
            pl.BlockSpec((_BLK, 1), lambda i: (i, 0)),
            pl.BlockSpec((1, D), lambda i: (0, 0)),
        ],
        out_specs=pl.BlockSpec((_BLK, D), lambda i: (i, 0)),
        out_shape=jax.ShapeDtypeStruct((N, D), jnp.float32),
    )(z, dinv, b)


def kernel(x, edge_index, W1, b1, W2, b2):
    src = edge_index[0].astype(jnp.int32)
    dst = edge_index[1].astype(jnp.int32)
    pad_idx = N + (jnp.arange(E_PAD - E, dtype=jnp.int32) % PAD_ROWS)

    def _perm(idx):
        l = idx % _BLK
        return (idx - l) + jnp.where(l < _HB, 2 * l, 2 * (l - _HB) + 1)

    srcp = _perm(jnp.concatenate([src, pad_idx])).reshape(NW * CH, K)
    dstp = _perm(jnp.concatenate([dst, pad_idx])).reshape(NW * CH, K)
    dst_deg = dst.reshape(E // K, K)
    x_pad = jnp.pad(x, ((0, N_PAD - N), (0, 0)))
    ones16 = jnp.ones((K, 16), jnp.float32)
    zeros16 = jnp.zeros((RPT, 16), jnp.float32)
    b1r = b1.reshape(1, D)
    b2r = b2.reshape(1, D)

    degp = _sc_degree(dst_deg, ones16, zeros16)
    y1, dinv = _tc_mm_scale(x_pad, W1, degp)
    z1 = _sc_aggregate(y1.reshape(NC, N_PAD, DH), srcp, dstp)
    y2 = _tc_mid(z1.reshape(NC, NP2, D), dinv, b1r, W2)
    z2 = _sc_aggregate(y2.reshape(NC, N_PAD, DH), srcp, dstp)
    return _tc_final(z2.reshape(NC, NP2, D), dinv, b2r)
